# Initial kernel scaffold; baseline (speedup 1.0000x reference)
import jax
import jax.numpy as jnp
from jax.experimental import pallas as pl


def kernel(features, edge_index, W1, b1, W2, b2):
    raise NotImplementedError("write your pallas kernel here")


# speedup vs baseline: 4.9713x; 4.9713x over previous
import functools

import jax
import jax.numpy as jnp
from jax import lax
from jax.experimental import pallas as pl
from jax.experimental.pallas import tpu as pltpu
from jax.experimental.pallas import tpu_sc as plsc

N = 10000
E = 320000
D1 = 128
D2 = 40

NC = 2
NS = 16
NW = NC * NS
E_PER_W = E // NW
CHUNK = 80
NCHUNK = E_PER_W // CHUNK
ROWS_PER_TILE = 624
REM_ROWS = N - NS * ROWS_PER_TILE


@functools.lru_cache(maxsize=None)
def _make_sc_segsum(D):
    mesh = plsc.VectorSubcoreMesh(core_axis_name="c", subcore_axis_name="s",
                                  num_cores=NC, num_subcores=NS)

    @functools.partial(
        pl.kernel,
        out_type=(
            jax.ShapeDtypeStruct((N, D), jnp.float32),
            jax.ShapeDtypeStruct((N, D), jnp.float32),
        ),
        mesh=mesh,
        compiler_params=pltpu.CompilerParams(use_tc_tiling_on_sc=False),
        scratch_types=[
            pltpu.VMEM((CHUNK,), jnp.int32),
            pltpu.VMEM((CHUNK,), jnp.int32),
            pltpu.VMEM((CHUNK, D), jnp.float32),
            pltpu.VMEM_SHARED((N, D), jnp.float32),
            pltpu.SemaphoreType.DMA,
        ],
    )
    def segsum(y_hbm, src_hbm, dst_hbm, zeros_hbm, out0, out1,
               src_c, dst_c, rows, acc, sem):
        c = lax.axis_index("c")
        s = lax.axis_index("s")
        wid = s * NC + c
        ebase = wid * E_PER_W

        rbase = s * ROWS_PER_TILE
        pltpu.sync_copy(zeros_hbm.at[pl.ds(rbase, ROWS_PER_TILE)],
                        acc.at[pl.ds(rbase, ROWS_PER_TILE)])

        @pl.when(s == 0)
        def _():
            pltpu.sync_copy(zeros_hbm.at[pl.ds(NS * ROWS_PER_TILE, REM_ROWS)],
                            acc.at[pl.ds(NS * ROWS_PER_TILE, REM_ROWS)])

        plsc.subcore_barrier()

        def body(i, carry):
            off = ebase + i * CHUNK
            pltpu.sync_copy(src_hbm.at[pl.ds(off, CHUNK)], src_c)
            pltpu.sync_copy(dst_hbm.at[pl.ds(off, CHUNK)], dst_c)
            pltpu.async_copy(y_hbm.at[src_c], rows, sem).wait()
            pltpu.sync_copy(rows, acc.at[dst_c], add=True)
            return carry

        lax.fori_loop(0, NCHUNK, body, 0)
        plsc.subcore_barrier()

        @pl.when(c == 0)
        def _():
            pltpu.sync_copy(acc.at[pl.ds(rbase, ROWS_PER_TILE)],
                            out0.at[pl.ds(rbase, ROWS_PER_TILE)])

            @pl.when(s == 0)
            def _():
                pltpu.sync_copy(acc.at[pl.ds(NS * ROWS_PER_TILE, REM_ROWS)],
                                out0.at[pl.ds(NS * ROWS_PER_TILE, REM_ROWS)])

        @pl.when(c == 1)
        def _():
            pltpu.sync_copy(acc.at[pl.ds(rbase, ROWS_PER_TILE)],
                            out1.at[pl.ds(rbase, ROWS_PER_TILE)])

            @pl.when(s == 0)
            def _():
                pltpu.sync_copy(acc.at[pl.ds(NS * ROWS_PER_TILE, REM_ROWS)],
                                out1.at[pl.ds(NS * ROWS_PER_TILE, REM_ROWS)])

    return segsum


_BLK = 1000


def _mm_body(x_ref, w_ref, o_ref):
    o_ref[...] = jnp.dot(x_ref[...], w_ref[...],
                         preferred_element_type=jnp.float32)


def _tc_matmul(x, w):
    m, k = x.shape
    n = w.shape[1]
    return pl.pallas_call(
        _mm_body,
        grid=(m // _BLK,),
        in_specs=[
            pl.BlockSpec((_BLK, k), lambda i: (i, 0)),
            pl.BlockSpec((k, n), lambda i: (0, 0)),
        ],
        out_specs=pl.BlockSpec((_BLK, n), lambda i: (i, 0)),
        out_shape=jax.ShapeDtypeStruct((m, n), jnp.float32),
    )(x, w)


def _fuse_body(y_ref, p0_ref, p1_ref, b_ref, w_ref, o_ref):
    z = y_ref[...] + p0_ref[...] + p1_ref[...] + b_ref[...]
    z = jnp.maximum(z, 0.0)
    o_ref[...] = jnp.dot(z, w_ref[...], preferred_element_type=jnp.float32)


def _tc_fuse_matmul(y, p0, p1, b, w):
    m, k = y.shape
    n = w.shape[1]
    return pl.pallas_call(
        _fuse_body,
        grid=(m // _BLK,),
        in_specs=[
            pl.BlockSpec((_BLK, k), lambda i: (i, 0)),
            pl.BlockSpec((_BLK, k), lambda i: (i, 0)),
            pl.BlockSpec((_BLK, k), lambda i: (i, 0)),
            pl.BlockSpec((1, k), lambda i: (0, 0)),
            pl.BlockSpec((k, n), lambda i: (0, 0)),
        ],
        out_specs=pl.BlockSpec((_BLK, n), lambda i: (i, 0)),
        out_shape=jax.ShapeDtypeStruct((m, n), jnp.float32),
    )(y, p0, p1, b, w)


def _final_body(y_ref, q0_ref, q1_ref, b_ref, o_ref):
    o_ref[...] = y_ref[...] + q0_ref[...] + q1_ref[...] + b_ref[...]


def _tc_final(y, q0, q1, b):
    m, n = y.shape
    return pl.pallas_call(
        _final_body,
        grid=(m // _BLK,),
        in_specs=[
            pl.BlockSpec((_BLK, n), lambda i: (i, 0)),
            pl.BlockSpec((_BLK, n), lambda i: (i, 0)),
            pl.BlockSpec((_BLK, n), lambda i: (i, 0)),
            pl.BlockSpec((1, n), lambda i: (0, 0)),
        ],
        out_specs=pl.BlockSpec((_BLK, n), lambda i: (i, 0)),
        out_shape=jax.ShapeDtypeStruct((m, n), jnp.float32),
    )(y, q0, q1, b)


def kernel(features, edge_index, W1, b1, W2, b2):
    src = edge_index[0]
    dst = edge_index[1]
    z1 = jnp.zeros((N, D1), jnp.float32)
    z2 = jnp.zeros((N, D2), jnp.float32)
    b1r = b1.reshape(1, D1)
    b2r = b2.reshape(1, D2)

    y1 = _tc_matmul(features, W1)
    p0, p1 = _make_sc_segsum(D1)(y1, src, dst, z1)
    y2 = _tc_fuse_matmul(y1, p0, p1, b1r, W2)
    q0, q1 = _make_sc_segsum(D2)(y2, src, dst, z2)
    return _tc_final(y2, q0, q1, b2r)


# speedup vs baseline: 13.9667x; 2.8095x over previous
import functools

import jax
import jax.numpy as jnp
from jax import lax
from jax.experimental import pallas as pl
from jax.experimental.pallas import tpu as pltpu
from jax.experimental.pallas import tpu_sc as plsc

N = 10000
E = 320000
D1 = 128
D2 = 40

NC = 2
NS = 16
NW = NC * NS
E_PER_W = E // NW
ROWS_PER_TILE = 624
REM_ROWS = N - NS * ROWS_PER_TILE

_CHUNK = {128: 40, 40: 80}
_NBUF = {128: 5, 40: 5}


@functools.lru_cache(maxsize=None)
def _make_sc_segsum(D):
    CHUNK = _CHUNK[D]
    NBUF = _NBUF[D]
    NCHUNK = E_PER_W // CHUNK
    mesh = plsc.VectorSubcoreMesh(core_axis_name="c", subcore_axis_name="s",
                                  num_cores=NC, num_subcores=NS)

    @functools.partial(
        pl.kernel,
        out_type=(
            jax.ShapeDtypeStruct((N, D), jnp.float32),
            jax.ShapeDtypeStruct((N, D), jnp.float32),
        ),
        mesh=mesh,
        compiler_params=pltpu.CompilerParams(use_tc_tiling_on_sc=False),
        scratch_types=[
            pltpu.VMEM((NCHUNK, CHUNK), jnp.int32),
            pltpu.VMEM((NCHUNK, CHUNK), jnp.int32),
            [pltpu.VMEM((CHUNK, D), jnp.float32) for _ in range(NBUF)],
            pltpu.VMEM_SHARED((N, D), jnp.float32),
            [pltpu.SemaphoreType.DMA for _ in range(NBUF)],
            [pltpu.SemaphoreType.DMA for _ in range(NBUF)],
            pltpu.SemaphoreType.DMA,
        ],
    )
    def segsum(y_hbm, src_hbm, dst_hbm, zeros_hbm, out0, out1,
               src_v, dst_v, rows, acc, gsem, ssem, sem):
        c = lax.axis_index("c")
        s = lax.axis_index("s")
        wid = s * NC + c
        cbase = wid * NCHUNK

        pltpu.async_copy(src_hbm.at[pl.ds(cbase, NCHUNK)], src_v, sem)
        pltpu.async_copy(dst_hbm.at[pl.ds(cbase, NCHUNK)], dst_v, sem)

        rbase = s * ROWS_PER_TILE
        pltpu.sync_copy(zeros_hbm.at[pl.ds(rbase, ROWS_PER_TILE)],
                        acc.at[pl.ds(rbase, ROWS_PER_TILE)])

        @pl.when(s == 0)
        def _():
            pltpu.sync_copy(zeros_hbm.at[pl.ds(NS * ROWS_PER_TILE, REM_ROWS)],
                            acc.at[pl.ds(NS * ROWS_PER_TILE, REM_ROWS)])

        pltpu.make_async_copy(src_hbm.at[pl.ds(cbase, NCHUNK)], src_v,
                              sem).wait()
        pltpu.make_async_copy(dst_hbm.at[pl.ds(cbase, NCHUNK)], dst_v,
                              sem).wait()
        plsc.subcore_barrier()

        for b in range(NBUF - 1):
            pltpu.async_copy(y_hbm.at[src_v.at[b]], rows[b], gsem[b])

        @pl.loop(0, NCHUNK, step=NBUF)
        def _(j):
            for b in range(NBUF):
                i = j + b
                pltpu.make_async_copy(y_hbm.at[src_v.at[i]], rows[b],
                                      gsem[b]).wait()
                pltpu.async_copy(rows[b], acc.at[dst_v.at[i]], ssem[b],
                                 add=True)
                nb = (b + NBUF - 1) % NBUF
                ni = i + NBUF - 1

                @pl.when(ni < NCHUNK)
                def _():
                    @pl.when(ni >= NBUF)
                    def _():
                        pltpu.make_async_copy(
                            rows[nb], acc.at[dst_v.at[ni - NBUF]],
                            ssem[nb]).wait()

                    pltpu.async_copy(y_hbm.at[src_v.at[ni]], rows[nb],
                                     gsem[nb])

        for b in range(NBUF):
            i = NCHUNK - NBUF + b
            pltpu.make_async_copy(rows[b], acc.at[dst_v.at[i]],
                                  ssem[b]).wait()
        plsc.subcore_barrier()

        @pl.when(c == 0)
        def _():
            pltpu.sync_copy(acc.at[pl.ds(rbase, ROWS_PER_TILE)],
                            out0.at[pl.ds(rbase, ROWS_PER_TILE)])

            @pl.when(s == 0)
            def _():
                pltpu.sync_copy(acc.at[pl.ds(NS * ROWS_PER_TILE, REM_ROWS)],
                                out0.at[pl.ds(NS * ROWS_PER_TILE, REM_ROWS)])

        @pl.when(c == 1)
        def _():
            pltpu.sync_copy(acc.at[pl.ds(rbase, ROWS_PER_TILE)],
                            out1.at[pl.ds(rbase, ROWS_PER_TILE)])

            @pl.when(s == 0)
            def _():
                pltpu.sync_copy(acc.at[pl.ds(NS * ROWS_PER_TILE, REM_ROWS)],
                                out1.at[pl.ds(NS * ROWS_PER_TILE, REM_ROWS)])

    return segsum


_BLK = 1000


def _mm_body(x_ref, w_ref, o_ref):
    o_ref[...] = jnp.dot(x_ref[...], w_ref[...],
                         preferred_element_type=jnp.float32)


def _tc_matmul(x, w):
    m, k = x.shape
    n = w.shape[1]
    return pl.pallas_call(
        _mm_body,
        grid=(m // _BLK,),
        in_specs=[
            pl.BlockSpec((_BLK, k), lambda i: (i, 0)),
            pl.BlockSpec((k, n), lambda i: (0, 0)),
        ],
        out_specs=pl.BlockSpec((_BLK, n), lambda i: (i, 0)),
        out_shape=jax.ShapeDtypeStruct((m, n), jnp.float32),
    )(x, w)


def _fuse_body(y_ref, p0_ref, p1_ref, b_ref, w_ref, o_ref):
    z = y_ref[...] + p0_ref[...] + p1_ref[...] + b_ref[...]
    z = jnp.maximum(z, 0.0)
    o_ref[...] = jnp.dot(z, w_ref[...], preferred_element_type=jnp.float32)


def _tc_fuse_matmul(y, p0, p1, b, w):
    m, k = y.shape
    n = w.shape[1]
    return pl.pallas_call(
        _fuse_body,
        grid=(m // _BLK,),
        in_specs=[
            pl.BlockSpec((_BLK, k), lambda i: (i, 0)),
            pl.BlockSpec((_BLK, k), lambda i: (i, 0)),
            pl.BlockSpec((_BLK, k), lambda i: (i, 0)),
            pl.BlockSpec((1, k), lambda i: (0, 0)),
            pl.BlockSpec((k, n), lambda i: (0, 0)),
        ],
        out_specs=pl.BlockSpec((_BLK, n), lambda i: (i, 0)),
        out_shape=jax.ShapeDtypeStruct((m, n), jnp.float32),
    )(y, p0, p1, b, w)


def _final_body(y_ref, q0_ref, q1_ref, b_ref, o_ref):
    o_ref[...] = y_ref[...] + q0_ref[...] + q1_ref[...] + b_ref[...]


def _tc_final(y, q0, q1, b):
    m, n = y.shape
    return pl.pallas_call(
        _final_body,
        grid=(m // _BLK,),
        in_specs=[
            pl.BlockSpec((_BLK, n), lambda i: (i, 0)),
            pl.BlockSpec((_BLK, n), lambda i: (i, 0)),
            pl.BlockSpec((_BLK, n), lambda i: (i, 0)),
            pl.BlockSpec((1, n), lambda i: (0, 0)),
        ],
        out_specs=pl.BlockSpec((_BLK, n), lambda i: (i, 0)),
        out_shape=jax.ShapeDtypeStruct((m, n), jnp.float32),
    )(y, q0, q1, b)


def kernel(features, edge_index, W1, b1, W2, b2):
    src1 = edge_index[0].reshape(E // _CHUNK[D1], _CHUNK[D1])
    dst1 = edge_index[1].reshape(E // _CHUNK[D1], _CHUNK[D1])
    src2 = edge_index[0].reshape(E // _CHUNK[D2], _CHUNK[D2])
    dst2 = edge_index[1].reshape(E // _CHUNK[D2], _CHUNK[D2])
    z1 = jnp.zeros((N, D1), jnp.float32)
    z2 = jnp.zeros((N, D2), jnp.float32)
    b1r = b1.reshape(1, D1)
    b2r = b2.reshape(1, D2)

    y1 = _tc_matmul(features, W1)
    p0, p1 = _make_sc_segsum(D1)(y1, src1, dst1, z1)
    y2 = _tc_fuse_matmul(y1, p0, p1, b1r, W2)
    q0, q1 = _make_sc_segsum(D2)(y2, src2, dst2, z2)
    return _tc_final(y2, q0, q1, b2r)
